# baseline (speedup 1.0000x reference)
import functools

import jax
import jax.numpy as jnp
from jax import lax
from jax.experimental import pallas as pl
from jax.experimental.pallas import tpu as pltpu
from jax.experimental.pallas import tpu_sc as plsc

B = 16384
S = 5
D = 128
C = 128
NC = 2
NS = 16
NW = NC * NS
R_IN = B // NW
R_NZ = B * S // NW
N_IN = R_IN // C
N_NZ = R_NZ // C

W = 2
NPAIR = 3
PSKEW = 2


def _body(iw, ow, nzw, iemb, oemb, o_in, o_out, o_nz,
          idx_i, idx_o, idx_n, big, *rest):
    gsems = rest[:NPAIR * W]
    wsems = rest[NPAIR * W:NPAIR * W + NPAIR]
    isems = rest[NPAIR * W + NPAIR:NPAIR * W + NPAIR + 3]
    wid = lax.axis_index("s") * NC + lax.axis_index("c")

    ic = pltpu.async_copy(iw.at[pl.ds(wid * R_IN, R_IN)], idx_i, isems[0])
    oc = pltpu.async_copy(ow.at[pl.ds(wid * R_IN, R_IN)], idx_o, isems[1])
    nc = pltpu.async_copy(nzw.at[pl.ds(wid * R_NZ, R_NZ)], idx_n, isems[2])
    idx_ready = {id(idx_i): ic, id(idx_o): oc, id(idx_n): nc}

    pa = [(idx_i, j * C, iemb, o_in, wid * R_IN + j * C)
          for j in range(0, N_IN, W)]
    pb = [(idx_o, j * C, oemb, o_out, wid * R_IN + j * C)
          for j in range(0, N_IN, W)]
    pn = [(idx_n, j * C, oemb, o_nz, wid * R_NZ + j * C)
          for j in range(0, N_NZ, W)]
    pairs = []
    while pa or pb or pn:
        if pa:
            pairs.append(pa.pop(0))
        if pb:
            pairs.append(pb.pop(0))
        for _ in range(3):
            if pn:
                pairs.append(pn.pop(0))
    np_ = len(pairs)

    g = [[None] * W for _ in range(NPAIR)]
    w = [None] * NPAIR

    def fire(p):
        r = p % NPAIR
        if w[r] is not None:
            w[r].wait()
            w[r] = None
        idx, ib, tab, _, _ = pairs[p]
        rdy = idx_ready.pop(id(idx), None)
        if rdy is not None:
            rdy.wait()
        for u in range(W):
            g[r][u] = pltpu.async_copy(
                tab.at[idx.at[pl.ds(ib + u * C, C)]],
                big.at[pl.ds((r * W + u) * C, C)], gsems[r * W + u])

    def drain(p):
        r = p % NPAIR
        for u in range(W):
            g[r][u].wait()
        _, _, _, oref, ob = pairs[p]
        w[r] = pltpu.async_copy(
            big.at[pl.ds(r * W * C, W * C)],
            oref.at[pl.ds(ob, W * C)], wsems[r])

    for p in range(np_ + PSKEW):
        if p < np_:
            fire(p)
        if p >= PSKEW:
            drain(p - PSKEW)
    for r in range(NPAIR):
        if w[r] is not None:
            w[r].wait()


@functools.partial(
    pl.kernel,
    out_type=(
        jax.ShapeDtypeStruct((B, D), jnp.float32),
        jax.ShapeDtypeStruct((B, D), jnp.float32),
        jax.ShapeDtypeStruct((S * B, D), jnp.float32),
    ),
    mesh=plsc.VectorSubcoreMesh(core_axis_name="c", subcore_axis_name="s"),
    scratch_types=[
        pltpu.VMEM((R_IN,), jnp.int32),
        pltpu.VMEM((R_IN,), jnp.int32),
        pltpu.VMEM((R_NZ,), jnp.int32),
        pltpu.VMEM((NPAIR * W * C, D), jnp.float32),
        *[pltpu.SemaphoreType.DMA for _ in range(NPAIR * W + NPAIR + 3)],
    ],
)
def _gather_kernel(*refs):
    _body(*refs)


def kernel(input_words, output_words, noise_words, inp_embed, out_embed):
    iw = input_words.astype(jnp.int32)
    ow = output_words.astype(jnp.int32)
    nz = jnp.transpose(noise_words.astype(jnp.int32)).reshape(B * S)
    o_in, o_out, o_nz = _gather_kernel(iw, ow, nz, inp_embed, out_embed)
    return (o_in, o_out, jnp.transpose(o_nz.reshape(S, B, D), (1, 0, 2)))
